# baseline (speedup 1.0000x reference)
import functools

import jax
import jax.numpy as jnp
from jax import lax
from jax.experimental import pallas as pl
from jax.experimental.pallas import tpu as pltpu
from jax.experimental.pallas import tpu_sc as plsc

NUM_CLASSES = 100000
DIM = 64
BATCH = 16384
ROWS = NUM_CLASSES + 1

_INFO = plsc.get_sparse_core_info()
_NC = _INFO.num_cores
_NS = _INFO.num_subcores
_NW = _NC * _NS
_J_PER_W = DIM // _NW
_NCHUNK = 4
_CHUNK = BATCH // _NCHUNK
_UNROLL = 8
_GROUPS = _CHUNK // 16


def _make_gather():
  mesh = plsc.VectorSubcoreMesh(core_axis_name="c", subcore_axis_name="s")

  @functools.partial(
      pl.kernel,
      mesh=mesh,
      out_type=jax.ShapeDtypeStruct((DIM, BATCH), jnp.float32),
      scratch_types=[
          pltpu.VMEM((ROWS,), jnp.float32),
          pltpu.VMEM((BATCH,), jnp.int32),
          pltpu.VMEM((2, _CHUNK), jnp.float32),
          pltpu.SemaphoreType.DMA,
          pltpu.SemaphoreType.DMA,
          pltpu.SemaphoreType.DMA,
      ],
      compiler_params=pltpu.CompilerParams(use_tc_tiling_on_sc=True,
                                           needs_layout_passes=False),
  )
  def gather_kernel(labels_hbm, table_t_hbm, out_t_hbm, row_v, lab_v, res_v,
                    row_sem, out_sem_a, out_sem_b):
    wid = lax.axis_index("s") * _NC + lax.axis_index("c")
    out_sems = (out_sem_a, out_sem_b)

    lab_copy = pltpu.async_copy(labels_hbm, lab_v, row_sem)
    row_copy = pltpu.async_copy(table_t_hbm.at[wid * _J_PER_W], row_v, row_sem)
    lab_copy.wait()
    row_copy.wait()

    out_copies = []
    for jj in range(_J_PER_W):
      j = wid * _J_PER_W + jj
      for c in range(_NCHUNK):
        buf = c % 2
        if len(out_copies) >= 2:
          out_copies[-2].wait()

        def body(g, carry):
          for u in range(_UNROLL):
            off = c * _CHUNK + (g * _UNROLL + u) * 16
            idx = lab_v[pl.ds(off, 16)]
            res_v[buf, pl.ds((g * _UNROLL + u) * 16, 16)] = (
                plsc.load_gather(row_v, [idx]))
          return carry

        lax.fori_loop(0, _GROUPS // _UNROLL, body, 0)
        out_copies.append(
            pltpu.async_copy(res_v.at[buf],
                             out_t_hbm.at[j, pl.ds(c * _CHUNK, _CHUNK)],
                             out_sems[buf]))
      if jj + 1 < _J_PER_W:
        pltpu.async_copy(table_t_hbm.at[j + 1], row_v, row_sem).wait()
    for copy in out_copies[-2:]:
      copy.wait()

  return gather_kernel


_gather = _make_gather()


@jax.jit
def kernel(labels, table):
  out_t = _gather(labels.astype(jnp.int32), table.T)
  return out_t.T


# speedup vs baseline: 1.2037x; 1.2037x over previous
import functools

import jax
import jax.numpy as jnp
from jax import lax
from jax.experimental import pallas as pl
from jax.experimental.pallas import tpu as pltpu
from jax.experimental.pallas import tpu_sc as plsc

NUM_CLASSES = 100000
DIM = 64
BATCH = 16384
ROWS = NUM_CLASSES + 1

_INFO = plsc.get_sparse_core_info()
_NC = _INFO.num_cores
_NS = _INFO.num_subcores
_NW = _NC * _NS
_J_PER_W = DIM // _NW
_CHUNK = BATCH // 2
_GROUPS = _CHUNK // 16
_UNROLL = 8


def _make_gather():
  mesh = plsc.VectorSubcoreMesh(core_axis_name="c", subcore_axis_name="s")

  @functools.partial(
      pl.kernel,
      mesh=mesh,
      out_type=jax.ShapeDtypeStruct((DIM, BATCH), jnp.float32),
      scratch_types=[
          pltpu.VMEM((ROWS,), jnp.float32),
          pltpu.VMEM((BATCH,), jnp.int32),
          pltpu.VMEM((_CHUNK,), jnp.float32),
          pltpu.SemaphoreType.DMA,
      ],
      compiler_params=pltpu.CompilerParams(use_tc_tiling_on_sc=True,
                                           needs_layout_passes=False),
  )
  def gather_kernel(labels_hbm, table_t_hbm, out_t_hbm, row_v, lab_v, res_v,
                    sem):
    wid = lax.axis_index("s") * _NC + lax.axis_index("c")
    pltpu.sync_copy(labels_hbm, lab_v)

    for jj in range(_J_PER_W):
      j = wid * _J_PER_W + jj
      pltpu.sync_copy(table_t_hbm.at[j], row_v)
      for c in range(2):

        def body(g, carry):
          for u in range(_UNROLL):
            off = (g * _UNROLL + u) * 16
            idx = lab_v[pl.ds(c * _CHUNK + off, 16)]
            res_v[pl.ds(off, 16)] = plsc.load_gather(row_v, [idx])
          return carry

        lax.fori_loop(0, _GROUPS // _UNROLL, body, 0)
        pltpu.sync_copy(res_v, out_t_hbm.at[j, pl.ds(c * _CHUNK, _CHUNK)])

  return gather_kernel


_gather = _make_gather()


@jax.jit
def kernel(labels, table):
  out_t = _gather(labels.astype(jnp.int32), table.T)
  return out_t.T


# speedup vs baseline: 1.2248x; 1.0175x over previous
import functools

import jax
import jax.numpy as jnp
from jax import lax
from jax.experimental import pallas as pl
from jax.experimental.pallas import tpu as pltpu
from jax.experimental.pallas import tpu_sc as plsc

NUM_CLASSES = 100000
DIM = 64
BATCH = 16384
ROWS = NUM_CLASSES + 1

_INFO = plsc.get_sparse_core_info()
_NC = _INFO.num_cores
_NS = _INFO.num_subcores
_NW = _NC * _NS
_J_PER_W = DIM // _NW
_NCHUNK = 4
_CHUNK = BATCH // _NCHUNK
_GROUPS = _CHUNK // 16
_UNROLL = 8


def _make_gather():
  mesh = plsc.VectorSubcoreMesh(core_axis_name="c", subcore_axis_name="s")

  @functools.partial(
      pl.kernel,
      mesh=mesh,
      out_type=jax.ShapeDtypeStruct((DIM, BATCH), jnp.float32),
      scratch_types=[
          pltpu.VMEM((ROWS,), jnp.float32),
          pltpu.VMEM((BATCH,), jnp.int32),
          pltpu.VMEM((_CHUNK,), jnp.float32),
          pltpu.VMEM((_CHUNK,), jnp.float32),
          pltpu.SemaphoreType.DMA,
          pltpu.SemaphoreType.DMA,
          pltpu.SemaphoreType.DMA,
      ],
      compiler_params=pltpu.CompilerParams(use_tc_tiling_on_sc=True,
                                           needs_layout_passes=False),
  )
  def gather_kernel(labels_hbm, table_t_hbm, out_t_hbm, row_v, lab_v, res_a,
                    res_b, sem, out_sem_a, out_sem_b):
    wid = lax.axis_index("s") * _NC + lax.axis_index("c")
    bufs = (res_a, res_b)
    out_sems = (out_sem_a, out_sem_b)
    pltpu.sync_copy(labels_hbm, lab_v)

    pending = [None, None]
    for jj in range(_J_PER_W):
      j = wid * _J_PER_W + jj
      pltpu.sync_copy(table_t_hbm.at[j], row_v)
      for c in range(_NCHUNK):
        p = c % 2
        res_v = bufs[p]
        if pending[p] is not None:
          pending[p].wait()

        def body(g, carry):
          for u in range(_UNROLL):
            off = (g * _UNROLL + u) * 16
            idx = lab_v[pl.ds(c * _CHUNK + off, 16)]
            res_v[pl.ds(off, 16)] = plsc.load_gather(row_v, [idx])
          return carry

        lax.fori_loop(0, _GROUPS // _UNROLL, body, 0)
        pending[p] = pltpu.async_copy(
            res_v, out_t_hbm.at[j, pl.ds(c * _CHUNK, _CHUNK)], out_sems[p])
    for p in range(2):
      if pending[p] is not None:
        pending[p].wait()

  return gather_kernel


_gather = _make_gather()


@jax.jit
def kernel(labels, table):
  out_t = _gather(labels.astype(jnp.int32), table.T)
  return out_t.T


# speedup vs baseline: 1.4549x; 1.1878x over previous
import functools

import jax
import jax.numpy as jnp
from jax import lax
from jax.experimental import pallas as pl
from jax.experimental.pallas import tpu as pltpu
from jax.experimental.pallas import tpu_sc as plsc

NUM_CLASSES = 100000
DIM = 64
BATCH = 16384
ROWS = NUM_CLASSES + 1

_INFO = plsc.get_sparse_core_info()
_NC = _INFO.num_cores
_NS = _INFO.num_subcores
_NW = _NC * _NS
_J_PER_W = DIM // _NW
_NCHUNK = 4
_CHUNK = BATCH // _NCHUNK
_GROUPS = _CHUNK // 16
_UNROLL = 8


def _make_gather():
  mesh = plsc.VectorSubcoreMesh(core_axis_name="c", subcore_axis_name="s")

  @functools.partial(
      pl.kernel,
      mesh=mesh,
      out_type=jax.ShapeDtypeStruct((DIM, BATCH), jnp.float32),
      scratch_types=[
          pltpu.VMEM((ROWS,), jnp.float32),
          pltpu.VMEM((BATCH,), jnp.int32),
          pltpu.VMEM((_CHUNK,), jnp.float32),
          pltpu.VMEM((_CHUNK,), jnp.float32),
          pltpu.SemaphoreType.DMA,
          pltpu.SemaphoreType.DMA,
          pltpu.SemaphoreType.DMA,
      ],
      compiler_params=pltpu.CompilerParams(use_tc_tiling_on_sc=True,
                                           needs_layout_passes=False),
  )
  def gather_kernel(labels_hbm, table_t_hbm, out_t_hbm, row_v, lab_v, res_a,
                    res_b, sem, out_sem_a, out_sem_b):
    wid = lax.axis_index("s") * _NC + lax.axis_index("c")
    bufs = (res_a, res_b)
    out_sems = (out_sem_a, out_sem_b)
    lab_copy = pltpu.async_copy(labels_hbm, lab_v, out_sem_a)

    pending = [None, None]
    for jj in range(_J_PER_W):
      j = wid * _J_PER_W + jj
      pltpu.sync_copy(table_t_hbm.at[j], row_v)
      if jj == 0:
        lab_copy.wait()
      for c in range(_NCHUNK):
        p = c % 2
        res_v = bufs[p]
        if pending[p] is not None:
          pending[p].wait()

        @plsc.parallel_loop(0, _GROUPS, step=1, unroll=_UNROLL)
        def _gather_body(g, res_v=res_v, c=c):
          off = g * 16
          idx = lab_v[pl.ds(c * _CHUNK + off, 16)]
          res_v[pl.ds(off, 16)] = plsc.load_gather(row_v, [idx])
        pending[p] = pltpu.async_copy(
            res_v, out_t_hbm.at[j, pl.ds(c * _CHUNK, _CHUNK)], out_sems[p])
    for p in range(2):
      if pending[p] is not None:
        pending[p].wait()

  return gather_kernel


_gather = _make_gather()


@jax.jit
def kernel(labels, table):
  out_t = _gather(labels.astype(jnp.int32), table.T)
  return out_t.T
